# Initial kernel scaffold; baseline (speedup 1.0000x reference)
import jax
import jax.numpy as jnp
from jax.experimental import pallas as pl


def kernel(hv, hc, vadj_rows, vadj_cols, vadj_values, cadj_rows, cadj_cols, cadj_values, params):
    raise NotImplementedError("write your pallas kernel here")


# speedup vs baseline: 3.5471x; 3.5471x over previous
import dataclasses
import functools

import jax
import jax.numpy as jnp
from jax import lax
from jax.experimental import pallas as pl
from jax.experimental.pallas import tpu as pltpu
from jax.experimental.pallas import tpu_sc as plsc

N = 10000
D = 128
E = 320000
CHUNK = 128
NCHUNKS = E // CHUNK
NW = 32
ROWBLK = 200
NROWBLK = N // ROWBLK
_PREC = lax.Precision.HIGHEST


def _sc_spmm_body(dense_hbm, cols_hbm, rows_hbm, vals_hbm, out_hbm,
                  acc, idx_v, ridx_v, val_v, gbuf, zbuf):
    cid = lax.axis_index("c")
    sid = lax.axis_index("s")
    wid = sid * 2 + cid

    @pl.loop(0, ROWBLK)
    def _(r):
        for l in range(D // 16):
            zbuf[r, pl.ds(l * 16, 16)] = jnp.zeros((16,), jnp.float32)

    nt = (NROWBLK - sid + 15) // 16

    @pl.loop(0, nt)
    def _(t):
        c = sid + 16 * t
        pltpu.sync_copy(zbuf, acc.at[pl.ds(c * ROWBLK, ROWBLK)])

    plsc.subcore_barrier()

    nk = (NCHUNKS - wid + (NW - 1)) // NW

    @pl.loop(0, nk)
    def _(k):
        g = wid + k * NW
        pltpu.sync_copy(cols_hbm.at[g], idx_v)
        pltpu.sync_copy(vals_hbm.at[g], val_v)
        pltpu.sync_copy(rows_hbm.at[g], ridx_v)
        pltpu.sync_copy(dense_hbm.at[idx_v.at[0]], gbuf)

        zero16 = jnp.zeros((16,), jnp.int32)

        @pl.loop(0, CHUNK)
        def _(e):
            bidx = jnp.full((16,), e, jnp.int32)
            vbc = plsc.load_gather(val_v, [zero16, bidx])
            for l in range(D // 16):
                sl = (e, pl.ds(l * 16, 16))
                gbuf[sl] = gbuf[sl] * vbc

        pltpu.sync_copy(gbuf, acc.at[ridx_v.at[0]], add=True)

    plsc.subcore_barrier()

    @pl.loop(0, nt)
    def _(t):
        c = sid + 16 * t
        pltpu.sync_copy(acc.at[pl.ds(c * ROWBLK, ROWBLK)],
                        out_hbm.at[cid, pl.ds(c * ROWBLK, ROWBLK)])


def _sc_spmm(dense, cols2d, rows2d, vals2d):
    mesh = plsc.VectorSubcoreMesh(core_axis_name="c", subcore_axis_name="s")
    cp = pltpu.CompilerParams()
    if "needs_layout_passes" in pltpu.CompilerParams.__dataclass_fields__:
        cp = dataclasses.replace(cp, needs_layout_passes=False)
    k = pl.kernel(
        _sc_spmm_body,
        out_type=jax.ShapeDtypeStruct((2, N, D), jnp.float32),
        mesh=mesh,
        compiler_params=cp,
        scratch_types=[
            pltpu.VMEM_SHARED((N, D), jnp.float32),
            pltpu.VMEM((1, CHUNK), jnp.int32),
            pltpu.VMEM((1, CHUNK), jnp.int32),
            pltpu.VMEM((1, CHUNK), jnp.float32),
            pltpu.VMEM((CHUNK, D), jnp.float32),
            pltpu.VMEM((ROWBLK, D), jnp.float32),
        ],
    )
    return k(dense, cols2d, rows2d, vals2d)


BM = 2000
NB = N // BM


def _msg_body(x_ref, w1_ref, b1_ref, w2_ref, b2_ref, o_ref):
    x = x_ref[...]
    h = jnp.maximum(
        jnp.dot(x, w1_ref[0], preferred_element_type=jnp.float32,
                precision=_PREC) + b1_ref[0], 0.0)
    o_ref[...] = jnp.maximum(
        jnp.dot(h, w2_ref[0], preferred_element_type=jnp.float32,
                precision=_PREC) + b2_ref[0], 0.0)


def _msg_mlp(h, p_pos, p_neg):
    w1 = jnp.stack([p_pos["W1"], p_neg["W1"]])
    b1 = jnp.stack([p_pos["b1"], p_neg["b1"]])[:, None, :]
    w2 = jnp.stack([p_pos["W2"], p_neg["W2"]])
    b2 = jnp.stack([p_pos["b2"], p_neg["b2"]])[:, None, :]
    return pl.pallas_call(
        _msg_body,
        grid=(2, NB),
        in_specs=[
            pl.BlockSpec((BM, D), lambda p, j: (j, 0)),
            pl.BlockSpec((1, D, D), lambda p, j: (p, 0, 0)),
            pl.BlockSpec((1, 1, D), lambda p, j: (p, 0, 0)),
            pl.BlockSpec((1, D, D), lambda p, j: (p, 0, 0)),
            pl.BlockSpec((1, 1, D), lambda p, j: (p, 0, 0)),
        ],
        out_specs=pl.BlockSpec((BM, D), lambda p, j: (p * NB + j, 0)),
        out_shape=jax.ShapeDtypeStruct((2 * N, D), jnp.float32),
    )(h, w1, b1, w2, b2)


def _upd_body(h_ref, m0_ref, m1_ref, w1h_ref, w1m_ref, b1_ref,
              w2_ref, b2_ref, o_ref):
    m = m0_ref[0] + m1_ref[0]
    h1 = jnp.maximum(
        jnp.dot(h_ref[...], w1h_ref[...], preferred_element_type=jnp.float32,
                precision=_PREC)
        + jnp.dot(m, w1m_ref[...], preferred_element_type=jnp.float32,
                  precision=_PREC)
        + b1_ref[...], 0.0)
    o_ref[...] = jnp.maximum(
        jnp.dot(h1, w2_ref[...], preferred_element_type=jnp.float32,
                precision=_PREC) + b2_ref[...], 0.0)


def _upd_mlp(h, parts, p):
    w1h = p["W1"][:D]
    w1m = p["W1"][D:]
    return pl.pallas_call(
        _upd_body,
        grid=(NB,),
        in_specs=[
            pl.BlockSpec((BM, D), lambda j: (j, 0)),
            pl.BlockSpec((1, BM, D), lambda j: (0, j, 0)),
            pl.BlockSpec((1, BM, D), lambda j: (1, j, 0)),
            pl.BlockSpec((D, D), lambda j: (0, 0)),
            pl.BlockSpec((D, D), lambda j: (0, 0)),
            pl.BlockSpec((1, D), lambda j: (0, 0)),
            pl.BlockSpec((D, D), lambda j: (0, 0)),
            pl.BlockSpec((1, D), lambda j: (0, 0)),
        ],
        out_specs=pl.BlockSpec((BM, D), lambda j: (j, 0)),
        out_shape=jax.ShapeDtypeStruct((N, D), jnp.float32),
    )(h, parts, parts, w1h, w1m, p["b1"][None, :], p["W2"], p["b2"][None, :])


def kernel(hv, hc, vadj_rows, vadj_cols, vadj_values,
           cadj_rows, cadj_cols, cadj_values, params):
    cat_c = _msg_mlp(hc, params["fmv_pos"], params["fmv_neg"])
    vcols = vadj_cols.astype(jnp.int32).reshape(NCHUNKS, 1, CHUNK)
    vrows = vadj_rows.astype(jnp.int32).reshape(NCHUNKS, 1, CHUNK)
    vvals = vadj_values.reshape(NCHUNKS, 1, CHUNK)
    mv_parts = _sc_spmm(cat_c, vcols, vrows, vvals)

    cat_v = _msg_mlp(hv, params["fmc_pos"], params["fmc_neg"])
    ccols = cadj_cols.astype(jnp.int32).reshape(NCHUNKS, 1, CHUNK)
    crows = cadj_rows.astype(jnp.int32).reshape(NCHUNKS, 1, CHUNK)
    cvals = cadj_values.reshape(NCHUNKS, 1, CHUNK)
    mc_parts = _sc_spmm(cat_v, ccols, crows, cvals)

    hv_out = _upd_mlp(hv, mv_parts, params["fuv"])
    hc_out = _upd_mlp(hc, mc_parts, params["fuc"])
    return (hv_out, hc_out)
